# baseline (device time: 1197726 ns/iter reference)
import jax
import jax.numpy as jnp
from jax import lax
from jax.experimental import pallas as pl
from jax.experimental.pallas import tpu as pltpu

N_DEV = 4
N_HOPS = 2 * (N_DEV - 1)
BLK = 512


def kernel(x, w_mat):
    m = x.shape[0]
    ksh = x.shape[1]
    n = w_mat.shape[1]
    chunk = m // N_DEV
    half = n // 2
    nblk = chunk // BLK

    def body(x_ref, w_ref, out_ref, comm_ref, p_ref, xv, wv, ov, acc_ref,
             tmp_ref, send_sems, recv_sems, local_sems):
        my = lax.axis_index("i")
        right = lax.rem(my + 1, N_DEV)
        left = lax.rem(my + N_DEV - 1, N_DEV)

        def mod4(v):
            return lax.rem(v + 2 * N_DEV, N_DEV)

        def send_idx(d, h):
            if h < 3:
                return mod4(my - h) if d == 0 else mod4(my + h)
            g = h - 3
            return mod4(my + 1 - g) if d == 0 else mod4(my - 1 + g)

        def accum_idx(d, s):
            return mod4(my - s - 1) if d == 0 else mod4(my + s + 1)

        def recv_idx(d, g):
            return mod4(my - g) if d == 0 else mod4(my + g)

        def mk_rdma(d, h, t, chunk_idx):
            col0 = 0 if d == 0 else half
            rows = pl.ds(chunk_idx * chunk + t * BLK, BLK)
            src = (p_ref if h == 0 else out_ref).at[rows, pl.ds(col0, half)]
            if h < 3:
                dst = comm_ref.at[d, h, pl.ds(t * BLK, BLK), :]
            else:
                dst = out_ref.at[rows, pl.ds(col0, half)]
            return pltpu.make_async_remote_copy(
                src_ref=src, dst_ref=dst,
                send_sem=send_sems.at[d, h, t],
                recv_sem=recv_sems.at[d, h, t],
                device_id=(right if d == 0 else left,),
                device_id_type=pl.DeviceIdType.MESH)

        sends = []

        def start_send(d, h, t):
            r = mk_rdma(d, h, t, send_idx(d, h))
            r.start()
            sends.append(r)

        _loaded = {"x": None}

        def tile(ckey, c, j, b):
            if _loaded["x"] != ckey:
                _loaded["x"] = ckey
                cp = pltpu.make_async_copy(
                    x_ref.at[pl.ds(c * chunk, chunk), :], xv,
                    local_sems.at[0])
                cp.start()
                cp.wait()
            rows = slice(b * BLK, (b + 1) * BLK)
            ov[...] = jnp.dot(xv[rows, :], wv[:, j * half:(j + 1) * half],
                              preferred_element_type=jnp.float32)
            st = pltpu.make_async_copy(
                ov,
                p_ref.at[pl.ds(c * chunk + b * BLK, BLK),
                         pl.ds(j * half, half)],
                local_sems.at[2])
            st.start()
            st.wait()

        def accum(d, s, t):
            col0 = 0 if d == 0 else half
            row0 = accum_idx(d, s) * chunk + t * BLK
            c_p = pltpu.make_async_copy(
                p_ref.at[pl.ds(row0, BLK), pl.ds(col0, half)],
                acc_ref, local_sems.at[0])
            c_c = pltpu.make_async_copy(
                comm_ref.at[d, s, pl.ds(t * BLK, BLK), :],
                tmp_ref, local_sems.at[1])
            c_p.start()
            c_c.start()
            c_p.wait()
            c_c.wait()
            acc_ref[...] = acc_ref[...] + tmp_ref[...]
            c_o = pltpu.make_async_copy(
                acc_ref, out_ref.at[pl.ds(row0, BLK), pl.ds(col0, half)],
                local_sems.at[2])
            c_o.start()
            c_o.wait()

        w_cp0 = pltpu.make_async_copy(
            w_ref.at[:, pl.ds(0, half)], wv.at[:, pl.ds(0, half)],
            local_sems.at[1])
        w_cp0.start()
        w_cp1 = pltpu.make_async_copy(
            w_ref.at[:, pl.ds(half, half)], wv.at[:, pl.ds(half, half)],
            local_sems.at[3])
        w_cp1.start()
        x_cp = pltpu.make_async_copy(
            x_ref.at[pl.ds(my * chunk, chunk), :], xv, local_sems.at[0])
        x_cp.start()
        _loaded["x"] = "my"

        barrier = pltpu.get_barrier_semaphore()
        for nbr in (left, right):
            pl.semaphore_signal(barrier, inc=1, device_id=(nbr,),
                                device_id_type=pl.DeviceIdType.MESH)
        pl.semaphore_wait(barrier, 2)
        x_cp.wait()
        w_cp0.wait()

        w1_waited = False
        for b in range(nblk):
            for j in (0, 1):
                if j == 1 and not w1_waited:
                    w_cp1.wait()
                    w1_waited = True
                tile("my", my, j, b)
                start_send(j, 0, b)

        for b in range(nblk):
            tile("my-1", mod4(my - 1), 0, b)
        for b in range(nblk):
            tile("my+1", mod4(my + 1), 1, b)

        for s in range(N_DEV - 1):
            for t in range(nblk):
                for d in (0, 1):
                    mk_rdma(d, s, t, accum_idx(d, s)).wait_recv()
                    accum(d, s, t)
                    start_send(d, s + 1, t)
            if s == 0:
                for j in (0, 1):
                    for b in range(nblk):
                        tile("my+2", mod4(my + 2), j, b)
            elif s == 1:
                for b in range(nblk):
                    tile("my+1", mod4(my + 1), 0, b)
                for b in range(nblk):
                    tile("my-1", mod4(my - 1), 1, b)

        for g in range(N_DEV - 1):
            h = 3 + g
            for t in range(nblk):
                for d in (0, 1):
                    mk_rdma(d, h, t, recv_idx(d, g)).wait_recv()
                    if h < N_HOPS - 1:
                        start_send(d, h + 1, t)

        for r in sends:
            r.wait_send()

    out, _, _ = pl.pallas_call(
        body,
        out_shape=(
            jax.ShapeDtypeStruct((m, n), jnp.float32),
            jax.ShapeDtypeStruct((2, N_DEV - 1, chunk, half), jnp.float32),
            jax.ShapeDtypeStruct((m, n), jnp.float32),
        ),
        in_specs=[pl.BlockSpec(memory_space=pl.ANY),
                  pl.BlockSpec(memory_space=pl.ANY)],
        out_specs=[pl.BlockSpec(memory_space=pl.ANY),
                   pl.BlockSpec(memory_space=pl.ANY),
                   pl.BlockSpec(memory_space=pl.ANY)],
        scratch_shapes=[
            pltpu.MemorySpace.VMEM((chunk, ksh), jnp.float32),
            pltpu.MemorySpace.VMEM((ksh, n), jnp.float32),
            pltpu.MemorySpace.VMEM((BLK, half), jnp.float32),
            pltpu.MemorySpace.VMEM((BLK, half), jnp.float32),
            pltpu.MemorySpace.VMEM((BLK, half), jnp.float32),
            pltpu.SemaphoreType.DMA((2, N_HOPS, 2)),
            pltpu.SemaphoreType.DMA((2, N_HOPS, 2)),
            pltpu.SemaphoreType.DMA((4,)),
        ],
        compiler_params=pltpu.CompilerParams(
            collective_id=0, vmem_limit_bytes=60 * 1024 * 1024),
    )(x, w_mat)
    return out


# device time: 1196580 ns/iter; 1.0010x vs baseline; 1.0010x over previous
import jax
import jax.numpy as jnp
from jax import lax
from jax.experimental import pallas as pl
from jax.experimental.pallas import tpu as pltpu

jax.config.update("jax_compilation_cache_dir", "/tmp/scband_jax_cache")
jax.config.update("jax_persistent_cache_min_compile_time_secs", 1.0)

N_DEV = 4
N_HOPS = 2 * (N_DEV - 1)
BLK = 512


def kernel(x, w_mat):
    m = x.shape[0]
    ksh = x.shape[1]
    n = w_mat.shape[1]
    chunk = m // N_DEV
    half = n // 2
    nblk = chunk // BLK

    def body(x_ref, w_ref, out_ref, comm_ref, p_ref, xv, wv, ov, acc_ref,
             tmp_ref, send_sems, recv_sems, local_sems):
        my = lax.axis_index("i")
        right = lax.rem(my + 1, N_DEV)
        left = lax.rem(my + N_DEV - 1, N_DEV)

        def mod4(v):
            return lax.rem(v + 2 * N_DEV, N_DEV)

        def send_idx(d, h):
            if h < 3:
                return mod4(my - h) if d == 0 else mod4(my + h)
            g = h - 3
            return mod4(my + 1 - g) if d == 0 else mod4(my - 1 + g)

        def accum_idx(d, s):
            return mod4(my - s - 1) if d == 0 else mod4(my + s + 1)

        def recv_idx(d, g):
            return mod4(my - g) if d == 0 else mod4(my + g)

        def mk_rdma(d, h, t, chunk_idx):
            col0 = 0 if d == 0 else half
            rows = pl.ds(chunk_idx * chunk + t * BLK, BLK)
            src = (p_ref if h == 0 else out_ref).at[rows, pl.ds(col0, half)]
            if h < 3:
                dst = comm_ref.at[d, h, pl.ds(t * BLK, BLK), :]
            else:
                dst = out_ref.at[rows, pl.ds(col0, half)]
            return pltpu.make_async_remote_copy(
                src_ref=src, dst_ref=dst,
                send_sem=send_sems.at[d, h, t],
                recv_sem=recv_sems.at[d, h, t],
                device_id=(right if d == 0 else left,),
                device_id_type=pl.DeviceIdType.MESH)

        sends = []

        def start_send(d, h, t):
            r = mk_rdma(d, h, t, send_idx(d, h))
            r.start()
            sends.append(r)

        _loaded = {"x": None}

        def tile(ckey, c, j, b):
            if _loaded["x"] != ckey:
                _loaded["x"] = ckey
                cp = pltpu.make_async_copy(
                    x_ref.at[pl.ds(c * chunk, chunk), :], xv,
                    local_sems.at[0])
                cp.start()
                cp.wait()
            rows = slice(b * BLK, (b + 1) * BLK)
            ov[...] = jnp.dot(xv[rows, :], wv[:, j * half:(j + 1) * half],
                              preferred_element_type=jnp.float32)
            st = pltpu.make_async_copy(
                ov,
                p_ref.at[pl.ds(c * chunk + b * BLK, BLK),
                         pl.ds(j * half, half)],
                local_sems.at[2])
            st.start()
            st.wait()

        def accum(d, s, t):
            col0 = 0 if d == 0 else half
            row0 = accum_idx(d, s) * chunk + t * BLK
            c_p = pltpu.make_async_copy(
                p_ref.at[pl.ds(row0, BLK), pl.ds(col0, half)],
                acc_ref, local_sems.at[0])
            c_c = pltpu.make_async_copy(
                comm_ref.at[d, s, pl.ds(t * BLK, BLK), :],
                tmp_ref, local_sems.at[1])
            c_p.start()
            c_c.start()
            c_p.wait()
            c_c.wait()
            acc_ref[...] = acc_ref[...] + tmp_ref[...]
            c_o = pltpu.make_async_copy(
                acc_ref, out_ref.at[pl.ds(row0, BLK), pl.ds(col0, half)],
                local_sems.at[2])
            c_o.start()
            c_o.wait()

        w_cp0 = pltpu.make_async_copy(
            w_ref.at[:, pl.ds(0, half)], wv.at[:, pl.ds(0, half)],
            local_sems.at[1])
        w_cp0.start()
        w_cp1 = pltpu.make_async_copy(
            w_ref.at[:, pl.ds(half, half)], wv.at[:, pl.ds(half, half)],
            local_sems.at[3])
        w_cp1.start()
        x_cp = pltpu.make_async_copy(
            x_ref.at[pl.ds(my * chunk, chunk), :], xv, local_sems.at[0])
        x_cp.start()
        _loaded["x"] = "my"

        barrier = pltpu.get_barrier_semaphore()
        for nbr in (left, right):
            pl.semaphore_signal(barrier, inc=1, device_id=(nbr,),
                                device_id_type=pl.DeviceIdType.MESH)
        pl.semaphore_wait(barrier, 2)
        x_cp.wait()
        w_cp0.wait()

        w1_waited = False
        for b in range(nblk):
            for j in (0, 1):
                if j == 1 and not w1_waited:
                    w_cp1.wait()
                    w1_waited = True
                tile("my", my, j, b)
                start_send(j, 0, b)

        for b in range(nblk):
            tile("my-1", mod4(my - 1), 0, b)
        for b in range(nblk):
            tile("my+1", mod4(my + 1), 1, b)

        for s in range(N_DEV - 1):
            for t in range(nblk):
                for d in (0, 1):
                    mk_rdma(d, s, t, accum_idx(d, s)).wait_recv()
                    accum(d, s, t)
                    start_send(d, s + 1, t)
            if s == 0:
                for j in (0, 1):
                    for b in range(nblk):
                        tile("my+2", mod4(my + 2), j, b)
            elif s == 1:
                for b in range(nblk):
                    tile("my+1", mod4(my + 1), 0, b)
                for b in range(nblk):
                    tile("my-1", mod4(my - 1), 1, b)

        for g in range(N_DEV - 1):
            h = 3 + g
            for t in range(nblk):
                for d in (0, 1):
                    mk_rdma(d, h, t, recv_idx(d, g)).wait_recv()
                    if h < N_HOPS - 1:
                        start_send(d, h + 1, t)

        for r in sends:
            r.wait_send()

    out, _, _ = pl.pallas_call(
        body,
        out_shape=(
            jax.ShapeDtypeStruct((m, n), jnp.float32),
            jax.ShapeDtypeStruct((2, N_DEV - 1, chunk, half), jnp.float32),
            jax.ShapeDtypeStruct((m, n), jnp.float32),
        ),
        in_specs=[pl.BlockSpec(memory_space=pl.ANY),
                  pl.BlockSpec(memory_space=pl.ANY)],
        out_specs=[pl.BlockSpec(memory_space=pl.ANY),
                   pl.BlockSpec(memory_space=pl.ANY),
                   pl.BlockSpec(memory_space=pl.ANY)],
        scratch_shapes=[
            pltpu.MemorySpace.VMEM((chunk, ksh), jnp.float32),
            pltpu.MemorySpace.VMEM((ksh, n), jnp.float32),
            pltpu.MemorySpace.VMEM((BLK, half), jnp.float32),
            pltpu.MemorySpace.VMEM((BLK, half), jnp.float32),
            pltpu.MemorySpace.VMEM((BLK, half), jnp.float32),
            pltpu.SemaphoreType.DMA((2, N_HOPS, 2)),
            pltpu.SemaphoreType.DMA((2, N_HOPS, 2)),
            pltpu.SemaphoreType.DMA((4,)),
        ],
        compiler_params=pltpu.CompilerParams(
            collective_id=0, vmem_limit_bytes=60 * 1024 * 1024),
    )(x, w_mat)
    return out
